# Initial kernel scaffold; baseline (speedup 1.0000x reference)
import jax
import jax.numpy as jnp
from jax.experimental import pallas as pl


def kernel(qkey, qval, qmask, mkey, mval, mmask):
    raise NotImplementedError("write your pallas kernel here")


# speedup vs baseline: 80.6821x; 80.6821x over previous
import functools
import math

import jax
import jax.numpy as jnp
from jax.experimental import pallas as pl
from jax.experimental.pallas import tpu as pltpu

_NEG = -3.0e38


def _flash_kernel(qk_ref, qv_ref, qm_ref, mk_ref, mv_ref, mm_ref, out_ref,
                  acc_ref, m_ref, l_ref, *, num_m_blocks, scale):
    j = pl.program_id(1)

    @pl.when(j == 0)
    def _init():
        acc_ref[...] = jnp.zeros_like(acc_ref)
        m_ref[...] = jnp.full_like(m_ref, _NEG)
        l_ref[...] = jnp.zeros_like(l_ref)

    qk = qk_ref[0]
    mk = mk_ref[0]
    mm = mm_ref[0, 0]

    s = jax.lax.dot_general(mk, qk, (((0,), (0,)), ((), ())),
                            preferred_element_type=jnp.float32) * scale
    s = jnp.where(mm[:, None] > 0.0, s, _NEG)

    m_prev = m_ref[...]
    m_new = jnp.maximum(m_prev, jnp.max(s, axis=0, keepdims=True))
    alpha = jnp.exp(m_prev - m_new)
    p = jnp.exp(s - m_new) * mm[:, None]
    l_ref[...] = l_ref[...] * alpha + jnp.sum(p, axis=0, keepdims=True)
    m_ref[...] = m_new

    mv = mv_ref[0]
    pv = jax.lax.dot_general(mv, p, (((1,), (0,)), ((), ())),
                             preferred_element_type=jnp.float32)
    acc_ref[...] = acc_ref[...] * alpha + pv

    @pl.when(j == num_m_blocks - 1)
    def _finish():
        l = l_ref[...]
        safe_l = jnp.where(l > 0.0, l, 1.0)
        read = acc_ref[...] / safe_l * (l > 0.0)
        qm = qm_ref[0]
        out_ref[0] = qv_ref[0] + read * qm


def kernel(qkey, qval, qmask, mkey, mval, mmask):
    B, Dk = qkey.shape[0], qkey.shape[1]
    Dv = mval.shape[1]
    qk = qkey.reshape(B, Dk, -1)
    qv = qval.reshape(B, Dv, -1)
    qm = qmask.reshape(B, 1, -1).astype(jnp.float32)
    mk = mkey.reshape(B, Dk, -1)
    mv = mval.reshape(B, Dv, -1)
    mm = mmask.reshape(B, 1, -1).astype(jnp.float32)
    Nq = qk.shape[-1]
    Nm = mk.shape[-1]

    BM = 1024
    num_m_blocks = Nm // BM
    scale = 1.0 / math.sqrt(Dk)

    grid = (B, num_m_blocks)
    out = pl.pallas_call(
        functools.partial(_flash_kernel, num_m_blocks=num_m_blocks,
                          scale=scale),
        grid=grid,
        in_specs=[
            pl.BlockSpec((1, Dk, Nq), lambda b, j: (b, 0, 0)),
            pl.BlockSpec((1, Dv, Nq), lambda b, j: (b, 0, 0)),
            pl.BlockSpec((1, 1, Nq), lambda b, j: (b, 0, 0)),
            pl.BlockSpec((1, Dk, BM), lambda b, j: (b, 0, j)),
            pl.BlockSpec((1, Dv, BM), lambda b, j: (b, 0, j)),
            pl.BlockSpec((1, 1, BM), lambda b, j: (b, 0, j)),
        ],
        out_specs=pl.BlockSpec((1, Dv, Nq), lambda b, j: (b, 0, 0)),
        out_shape=jax.ShapeDtypeStruct((B, Dv, Nq), jnp.float32),
        scratch_shapes=[
            pltpu.VMEM((Dv, Nq), jnp.float32),
            pltpu.VMEM((1, Nq), jnp.float32),
            pltpu.VMEM((1, Nq), jnp.float32),
        ],
        compiler_params=pltpu.CompilerParams(
            dimension_semantics=("parallel", "arbitrary"),
        ),
    )(qk, qv, qm, mk, mv, mm)
    return out.reshape(qval.shape)
